# baseline (speedup 1.0000x reference)
import dataclasses
import functools

import jax
import jax.numpy as jnp
from jax import lax
from jax.experimental import pallas as pl
from jax.experimental.pallas import tpu as pltpu
from jax.experimental.pallas import tpu_sc as plsc

N = 10000
E = 160000
IN_C = 128
HID = 8

NC = 2
NS = 16
NW = NC * NS
CH = 1024
BE = 3200

EA = 76800
EB = 83200
NFA, NTA = 75, 0
NFB, NTB = 81, 2

_mesh = lambda: plsc.VectorSubcoreMesh(core_axis_name="c", subcore_axis_name="s")


def _sc_params():
    cp = pltpu.CompilerParams()
    if "needs_layout_passes" in pltpu.CompilerParams.__dataclass_fields__:
        cp = dataclasses.replace(cp, needs_layout_passes=False)
    return cp


def _sc_gather_x(table, idx, start, count, chunk=200):
    _, d = table.shape
    per_w = count // NW
    n_chunks = per_w // chunk
    npairs = n_chunks // 2

    @functools.partial(
        pl.kernel,
        mesh=_mesh(),
        out_type=jax.ShapeDtypeStruct((count, d), table.dtype),
        scratch_types=[
            pltpu.VMEM((per_w,), jnp.int32),
            pltpu.VMEM((chunk, d), table.dtype),
            pltpu.VMEM((chunk, d), table.dtype),
            pltpu.SemaphoreType.DMA,
            pltpu.SemaphoreType.DMA,
            pltpu.SemaphoreType.DMA,
            pltpu.SemaphoreType.DMA,
        ],
    )
    def k(table_hbm, idx_hbm, out_hbm, idx_v, buf0, buf1, gs0, gs1, ws0, ws1):
        wid = lax.axis_index("s") * NC + lax.axis_index("c")
        base = wid * per_w
        pltpu.sync_copy(idx_hbm.at[pl.ds(start + base, per_w)], idx_v)

        def gsrc(j):
            return table_hbm.at[idx_v.at[pl.ds(j * chunk, chunk)]]

        def odst(j):
            return out_hbm.at[pl.ds(base + j * chunk, chunk)]

        pltpu.make_async_copy(gsrc(0), buf0, gs0).start()
        pltpu.make_async_copy(gsrc(1), buf1, gs1).start()

        @pl.loop(0, npairs)
        def _(p):
            j0 = 2 * p
            pltpu.make_async_copy(gsrc(j0), buf0, gs0).wait()
            pltpu.make_async_copy(buf0, odst(j0), ws0).start()
            pltpu.make_async_copy(gsrc(j0 + 1), buf1, gs1).wait()
            pltpu.make_async_copy(buf1, odst(j0 + 1), ws1).start()
            pltpu.make_async_copy(buf0, odst(j0), ws0).wait()

            @pl.when(j0 + 2 < n_chunks)
            def _():
                pltpu.make_async_copy(gsrc(j0 + 2), buf0, gs0).start()

            pltpu.make_async_copy(buf1, odst(j0 + 1), ws1).wait()

            @pl.when(j0 + 3 < n_chunks)
            def _():
                pltpu.make_async_copy(gsrc(j0 + 3), buf1, gs1).start()

        if n_chunks % 2:
            j = n_chunks - 1
            pltpu.make_async_copy(gsrc(j), buf0, gs0).wait()
            pltpu.sync_copy(buf0, odst(j))

    return k(table, idx)


def _half_chunks(wid, n_full, n_tail, chunk_fn):
    nloop = (n_full + NW - 1) // NW

    @pl.loop(0, nloop)
    def _(j):
        c = wid + j * NW

        @pl.when(c < n_full)
        def _():
            chunk_fn(c * CH, CH)

    for t in range(n_tail):
        @pl.when(wid == 28 + t)
        def _():
            chunk_fn(n_full * CH + t * 128, 128)


def _sc_gather_h1(h1t, idx, start, count, n_full, n_tail):
    @functools.partial(
        pl.kernel,
        mesh=_mesh(),
        compiler_params=_sc_params(),
        out_type=jax.ShapeDtypeStruct((HID, count), jnp.float32),
        scratch_types=[
            pltpu.VMEM((HID, N), jnp.float32),
            pltpu.VMEM((CH,), jnp.int32),
            pltpu.VMEM((HID, CH), jnp.float32),
        ],
    )
    def k(h1_hbm, idx_hbm, out_hbm, h1_v, idx_v, out_v):
        wid = lax.axis_index("s") * NC + lax.axis_index("c")
        pltpu.sync_copy(h1_hbm, h1_v)
        rows = [jnp.full((16,), i, jnp.int32) for i in range(HID)]

        def chunk(off, n_edges):
            pltpu.sync_copy(idx_hbm.at[pl.ds(start + off, n_edges)],
                            idx_v.at[pl.ds(0, n_edges)])

            @pl.loop(0, n_edges // 16)
            def _(g):
                srcv = idx_v[pl.ds(g * 16, 16)]
                for i in range(HID):
                    vals = plsc.load_gather(h1_v, [rows[i], srcv])
                    out_v[i, pl.ds(g * 16, 16)] = vals

            pltpu.sync_copy(out_v.at[:, pl.ds(0, n_edges)],
                            out_hbm.at[:, pl.ds(off, n_edges)])

        _half_chunks(wid, n_full, n_tail, chunk)

    return k(h1t, idx)


def _sc_segsum(mt, dst, zeros8n, with_count, start, n_full, n_tail):
    outs = [jax.ShapeDtypeStruct((NW, HID, N), jnp.float32)]
    if with_count:
        outs.append(jax.ShapeDtypeStruct((NW, N), jnp.float32))

    @functools.partial(
        pl.kernel,
        mesh=_mesh(),
        compiler_params=_sc_params(),
        out_type=tuple(outs) if with_count else outs[0],
        scratch_types=[
            pltpu.VMEM((HID, N), jnp.float32),
            pltpu.VMEM((N,), jnp.float32),
            pltpu.VMEM((CH,), jnp.int32),
            pltpu.VMEM((HID, CH), jnp.float32),
        ],
    )
    def k(m_hbm, dst_hbm, z_hbm, *out_and_scratch):
        if with_count:
            acc_hbm, cnt_hbm, acc_v, cnt_v, dst_v, m_v = out_and_scratch
        else:
            acc_hbm, acc_v, cnt_v, dst_v, m_v = out_and_scratch
        wid = lax.axis_index("s") * NC + lax.axis_index("c")
        pltpu.sync_copy(z_hbm, acc_v)
        if with_count:
            pltpu.sync_copy(z_hbm.at[0], cnt_v)
        rows = [jnp.full((16,), o, jnp.int32) for o in range(HID)]
        ones = jnp.full((16,), 1.0, jnp.float32)

        def chunk(off, n_edges):
            pltpu.sync_copy(dst_hbm.at[pl.ds(start + off, n_edges)],
                            dst_v.at[pl.ds(0, n_edges)])
            pltpu.sync_copy(m_hbm.at[:, pl.ds(off, n_edges)],
                            m_v.at[:, pl.ds(0, n_edges)])

            @pl.loop(0, n_edges // 16)
            def _(g):
                dstv = dst_v[pl.ds(g * 16, 16)]
                if with_count:
                    plsc.addupdate_scatter(cnt_v, [dstv], ones)
                for o in range(HID):
                    vals = m_v[o, pl.ds(g * 16, 16)]
                    plsc.addupdate_scatter(acc_v, [rows[o], dstv], vals)

        _half_chunks(wid, n_full, n_tail, chunk)

        pltpu.sync_copy(acc_v, acc_hbm.at[wid])
        if with_count:
            pltpu.sync_copy(cnt_v, cnt_hbm.at[wid])

    return k(mt, dst, zeros8n)


def _msg1_body(ea_ref, xg_ref, w1at_ref, b1a_ref, wb2_ref, st_ref, bb2t_ref,
               out_ref):
    h = jnp.maximum(
        jnp.dot(ea_ref[...], w1at_ref[...], preferred_element_type=jnp.float32)
        + b1a_ref[...], 0.0)
    xg = xg_ref[...]
    y = jnp.dot(xg.astype(jnp.bfloat16), wb2_ref[...],
                preferred_element_type=jnp.float32)
    prod = (y * jnp.tile(h, (1, HID))).astype(jnp.bfloat16)
    mt = lax.dot_general(st_ref[...], prod, (((1,), (1,)), ((), ())),
                         preferred_element_type=jnp.float32)
    cbt = lax.dot_general(bb2t_ref[...], xg, (((1,), (1,)), ((), ())),
                          preferred_element_type=jnp.float32)
    out_ref[...] = mt + cbt


def _msg2_body(ea_ref, hgt_ref, w2at_ref, b2a_ref, wp2_ref, st_ref, b2t_ref,
               out_ref):
    h2 = jnp.maximum(
        jnp.dot(ea_ref[...], w2at_ref[...], preferred_element_type=jnp.float32)
        + b2a_ref[...], 0.0)
    hgt = hgt_ref[...]
    q = lax.dot_general(hgt.astype(jnp.bfloat16), wp2_ref[...],
                        (((0,), (0,)), ((), ())),
                        preferred_element_type=jnp.float32)
    prod = (q * jnp.tile(h2, (1, HID))).astype(jnp.bfloat16)
    mt = lax.dot_general(st_ref[...], prod, (((1,), (1,)), ((), ())),
                         preferred_element_type=jnp.float32)
    c2t = lax.dot_general(b2t_ref[...], hgt, (((1,), (0,)), ((), ())),
                          preferred_element_type=jnp.float32)
    out_ref[...] = mt + c2t


def _node1_body(pa_ref, pb_ref, ca_ref, cb_ref, xt_ref, r1_ref, b1_ref,
                out_ref):
    s = jnp.sum(pa_ref[...], axis=0) + jnp.sum(pb_ref[...], axis=0)
    cnt = jnp.sum(ca_ref[...], axis=0) + jnp.sum(cb_ref[...], axis=0)
    mean = s / jnp.maximum(cnt, 1.0)[None, :]
    xrt = jnp.dot(r1_ref[...], xt_ref[...], preferred_element_type=jnp.float32)
    out_ref[...] = jnp.maximum(mean + xrt + b1_ref[...], 0.0)


def _node2_body(pa_ref, pb_ref, ca_ref, cb_ref, h1t_ref, r2_ref, b2_ref,
                wl1_ref, bl1_ref, wl2_ref, bl2_ref, out_ref):
    s = jnp.sum(pa_ref[...], axis=0) + jnp.sum(pb_ref[...], axis=0)
    cnt = jnp.sum(ca_ref[...], axis=0) + jnp.sum(cb_ref[...], axis=0)
    mean = s / jnp.maximum(cnt, 1.0)[None, :]
    h2 = jnp.maximum(
        mean + jnp.dot(r2_ref[...], h1t_ref[...],
                       preferred_element_type=jnp.float32) + b2_ref[...], 0.0)
    h3 = jnp.maximum(
        jnp.dot(wl1_ref[...], h2, preferred_element_type=jnp.float32)
        + bl1_ref[...], 0.0)
    out_ref[...] = jnp.dot(wl2_ref[...], h3,
                           preferred_element_type=jnp.float32) + bl2_ref[...]


def _full(shape):
    nd = len(shape)
    return pl.BlockSpec(shape, lambda i, _n=nd: (0,) * _n)


def kernel(x, edge_index, edge_attr, W1a, b1a, W1b, b1b, root1, bias1,
           W2a, b2a, W2b, b2b, root2, bias2, Wl1, bl1, Wl2, bl2):
    src = edge_index[0]
    dst = edge_index[1]
    zeros8n = jnp.zeros((HID, N), jnp.float32)
    sel = jnp.kron(jnp.eye(HID, dtype=jnp.float32),
                   jnp.ones((1, 64), jnp.float32)).astype(jnp.bfloat16)

    w1at = W1a.T
    wb2 = W1b.reshape(IN_C, HID, 64).reshape(IN_C, HID * 64)
    wb2 = wb2.astype(jnp.bfloat16)
    bb2t = b1b.reshape(IN_C, HID).T
    w2at = W2a.T
    wp2 = W2b.reshape(HID, HID * 64).astype(jnp.bfloat16)
    b2t = b2b.reshape(HID, HID).T
    xt = x.T

    def msg1(xg_half, count, blk_off):
        return pl.pallas_call(
            _msg1_body,
            grid=(count // BE,),
            in_specs=[
                pl.BlockSpec((BE, 4), lambda i: (i + blk_off, 0)),
                pl.BlockSpec((BE, IN_C), lambda i: (i, 0)),
                _full((4, 64)), _full((1, 64)),
                _full((IN_C, HID * 64)), _full((HID, HID * 64)),
                _full((HID, IN_C)),
            ],
            out_specs=pl.BlockSpec((HID, BE), lambda i: (0, i)),
            out_shape=jax.ShapeDtypeStruct((HID, count), jnp.float32),
        )(edge_attr, xg_half, w1at, b1a.reshape(1, 64), wb2, sel, bb2t)

    def msg2(hgt_half, count, blk_off):
        return pl.pallas_call(
            _msg2_body,
            grid=(count // BE,),
            in_specs=[
                pl.BlockSpec((BE, 4), lambda i: (i + blk_off, 0)),
                pl.BlockSpec((HID, BE), lambda i: (0, i)),
                _full((4, 64)), _full((1, 64)),
                _full((HID, HID * 64)), _full((HID, HID * 64)),
                _full((HID, HID)),
            ],
            out_specs=pl.BlockSpec((HID, BE), lambda i: (0, i)),
            out_shape=jax.ShapeDtypeStruct((HID, count), jnp.float32),
        )(edge_attr, hgt_half, w2at, b2a.reshape(1, 64), wp2, sel, b2t)

    xga = _sc_gather_x(x, src, 0, EA)
    xgb = _sc_gather_x(x, src, EA, EB)
    m1a = msg1(xga, EA, 0)
    m1b = msg1(xgb, EB, EA // BE)
    p1a, cnta = _sc_segsum(m1a, dst, zeros8n, True, 0, NFA, NTA)
    p1b, cntb = _sc_segsum(m1b, dst, zeros8n, True, EA, NFB, NTB)

    h1t = pl.pallas_call(
        _node1_body,
        out_shape=jax.ShapeDtypeStruct((HID, N), jnp.float32),
    )(p1a, p1b, cnta, cntb, xt, root1, bias1.reshape(HID, 1))

    hga = _sc_gather_h1(h1t, src, 0, EA, NFA, NTA)
    hgb = _sc_gather_h1(h1t, src, EA, EB, NFB, NTB)
    m2a = msg2(hga, EA, 0)
    m2b = msg2(hgb, EB, EA // BE)
    p2a = _sc_segsum(m2a, dst, zeros8n, False, 0, NFA, NTA)
    p2b = _sc_segsum(m2b, dst, zeros8n, False, EA, NFB, NTB)

    out = pl.pallas_call(
        _node2_body,
        out_shape=jax.ShapeDtypeStruct((1, N), jnp.float32),
    )(p2a, p2b, cnta, cntb, h1t, root2, bias2.reshape(HID, 1),
      Wl1, bl1.reshape(8, 1), Wl2, bl2.reshape(1, 1))

    return out[0]


# speedup vs baseline: 1.0032x; 1.0032x over previous
import dataclasses
import functools

import jax
import jax.numpy as jnp
from jax import lax
from jax.experimental import pallas as pl
from jax.experimental.pallas import tpu as pltpu
from jax.experimental.pallas import tpu_sc as plsc

N = 10000
E = 160000
IN_C = 128
HID = 8

NC = 2
NS = 16
NW = NC * NS
CH = 1024
NFULL = E // CH
TAIL0 = NFULL * CH

_mesh = lambda: plsc.VectorSubcoreMesh(core_axis_name="c", subcore_axis_name="s")


def _sc_params():
    cp = pltpu.CompilerParams()
    if "needs_layout_passes" in pltpu.CompilerParams.__dataclass_fields__:
        cp = dataclasses.replace(cp, needs_layout_passes=False)
    return cp


def _sc_gather_x(table, idx, chunk=200):
    num, (_, d) = idx.shape[0], table.shape
    per_w = num // NW
    n_chunks = per_w // chunk
    npairs = n_chunks // 2

    @functools.partial(
        pl.kernel,
        mesh=_mesh(),
        out_type=jax.ShapeDtypeStruct((num, d), table.dtype),
        scratch_types=[
            pltpu.VMEM((per_w,), jnp.int32),
            pltpu.VMEM((chunk, d), table.dtype),
            pltpu.VMEM((chunk, d), table.dtype),
            pltpu.SemaphoreType.DMA,
            pltpu.SemaphoreType.DMA,
            pltpu.SemaphoreType.DMA,
            pltpu.SemaphoreType.DMA,
        ],
    )
    def k(table_hbm, idx_hbm, out_hbm, idx_v, buf0, buf1, gs0, gs1, ws0, ws1):
        wid = lax.axis_index("s") * NC + lax.axis_index("c")
        base = wid * per_w
        pltpu.sync_copy(idx_hbm.at[pl.ds(base, per_w)], idx_v)

        def gsrc(j):
            return table_hbm.at[idx_v.at[pl.ds(j * chunk, chunk)]]

        def odst(j):
            return out_hbm.at[pl.ds(base + j * chunk, chunk)]

        pltpu.make_async_copy(gsrc(0), buf0, gs0).start()
        pltpu.make_async_copy(gsrc(1), buf1, gs1).start()

        @pl.loop(0, npairs)
        def _(p):
            j0 = 2 * p
            pltpu.make_async_copy(gsrc(j0), buf0, gs0).wait()
            pltpu.make_async_copy(buf0, odst(j0), ws0).start()
            pltpu.make_async_copy(gsrc(j0 + 1), buf1, gs1).wait()
            pltpu.make_async_copy(buf1, odst(j0 + 1), ws1).start()
            pltpu.make_async_copy(buf0, odst(j0), ws0).wait()

            @pl.when(j0 + 2 < n_chunks)
            def _():
                pltpu.make_async_copy(gsrc(j0 + 2), buf0, gs0).start()

            pltpu.make_async_copy(buf1, odst(j0 + 1), ws1).wait()

            @pl.when(j0 + 3 < n_chunks)
            def _():
                pltpu.make_async_copy(gsrc(j0 + 3), buf1, gs1).start()

        if n_chunks % 2:
            j = n_chunks - 1
            pltpu.make_async_copy(gsrc(j), buf0, gs0).wait()
            pltpu.sync_copy(buf0, odst(j))

    return k(table, idx)


def _sc_gather_h1(h1t, idx):
    @functools.partial(
        pl.kernel,
        mesh=_mesh(),
        compiler_params=_sc_params(),
        out_type=jax.ShapeDtypeStruct((HID, E), jnp.float32),
        scratch_types=[
            pltpu.VMEM((HID, N), jnp.float32),
            pltpu.VMEM((CH,), jnp.int32),
            pltpu.VMEM((HID, CH), jnp.float32),
        ],
    )
    def k(h1_hbm, idx_hbm, out_hbm, h1_v, idx_v, out_v):
        wid = lax.axis_index("s") * NC + lax.axis_index("c")
        pltpu.sync_copy(h1_hbm, h1_v)
        rows = [jnp.full((16,), i, jnp.int32) for i in range(HID)]

        def chunk(off, n_edges):
            pltpu.sync_copy(idx_hbm.at[pl.ds(off, n_edges)],
                            idx_v.at[pl.ds(0, n_edges)])

            @pl.loop(0, n_edges // 16)
            def _(g):
                srcv = idx_v[pl.ds(g * 16, 16)]
                for i in range(HID):
                    vals = plsc.load_gather(h1_v, [rows[i], srcv])
                    out_v[i, pl.ds(g * 16, 16)] = vals

            pltpu.sync_copy(out_v.at[:, pl.ds(0, n_edges)],
                            out_hbm.at[:, pl.ds(off, n_edges)])

        @pl.loop(0, 5)
        def _(j):
            c = wid + j * NW

            @pl.when(c < NFULL)
            def _():
                chunk(c * CH, CH)

        @pl.when(wid == 28)
        def _():
            chunk(TAIL0, 128)

        @pl.when(wid == 29)
        def _():
            chunk(TAIL0 + 128, 128)

    return k(h1t, idx)


def _sc_segsum(mt, dst, zeros8n, with_count):
    outs = [jax.ShapeDtypeStruct((NW, HID, N), jnp.float32)]
    if with_count:
        outs.append(jax.ShapeDtypeStruct((NW, N), jnp.float32))

    @functools.partial(
        pl.kernel,
        mesh=_mesh(),
        compiler_params=_sc_params(),
        out_type=tuple(outs) if with_count else outs[0],
        scratch_types=[
            pltpu.VMEM((HID, N), jnp.float32),
            pltpu.VMEM((N,), jnp.float32),
            pltpu.VMEM((CH,), jnp.int32),
            pltpu.VMEM((HID, CH), jnp.float32),
        ],
    )
    def k(m_hbm, dst_hbm, z_hbm, *out_and_scratch):
        if with_count:
            acc_hbm, cnt_hbm, acc_v, cnt_v, dst_v, m_v = out_and_scratch
        else:
            acc_hbm, acc_v, cnt_v, dst_v, m_v = out_and_scratch
        wid = lax.axis_index("s") * NC + lax.axis_index("c")
        pltpu.sync_copy(z_hbm, acc_v)
        if with_count:
            pltpu.sync_copy(z_hbm.at[0], cnt_v)
        rows = [jnp.full((16,), o, jnp.int32) for o in range(HID)]
        ones = jnp.full((16,), 1.0, jnp.float32)

        def chunk(off, n_edges):
            pltpu.sync_copy(dst_hbm.at[pl.ds(off, n_edges)],
                            dst_v.at[pl.ds(0, n_edges)])
            pltpu.sync_copy(m_hbm.at[:, pl.ds(off, n_edges)],
                            m_v.at[:, pl.ds(0, n_edges)])

            @pl.loop(0, n_edges // 16)
            def _(g):
                dstv = dst_v[pl.ds(g * 16, 16)]
                if with_count:
                    plsc.addupdate_scatter(cnt_v, [dstv], ones)
                for o in range(HID):
                    vals = m_v[o, pl.ds(g * 16, 16)]
                    plsc.addupdate_scatter(acc_v, [rows[o], dstv], vals)

        @pl.loop(0, 5)
        def _(j):
            c = wid + j * NW

            @pl.when(c < NFULL)
            def _():
                chunk(c * CH, CH)

        @pl.when(wid == 28)
        def _():
            chunk(TAIL0, 128)

        @pl.when(wid == 29)
        def _():
            chunk(TAIL0 + 128, 128)

        pltpu.sync_copy(acc_v, acc_hbm.at[wid])
        if with_count:
            pltpu.sync_copy(cnt_v, cnt_hbm.at[wid])

    return k(mt, dst, zeros8n)


def _msg1_body(ea_ref, xg_ref, w1at_ref, b1a_ref, wb2_ref, st_ref, bb2t_ref,
               out_ref):
    h = jnp.maximum(
        jnp.dot(ea_ref[...], w1at_ref[...], preferred_element_type=jnp.float32)
        + b1a_ref[...], 0.0)
    xg = xg_ref[...]
    y = jnp.dot(xg.astype(jnp.bfloat16), wb2_ref[...],
                preferred_element_type=jnp.float32)
    prod = (y * jnp.tile(h, (1, HID))).astype(jnp.bfloat16)
    mt = lax.dot_general(st_ref[...], prod, (((1,), (1,)), ((), ())),
                         preferred_element_type=jnp.float32)
    cbt = lax.dot_general(bb2t_ref[...], xg, (((1,), (1,)), ((), ())),
                          preferred_element_type=jnp.float32)
    out_ref[...] = mt + cbt


def _msg2_body(ea_ref, hgt_ref, w2at_ref, b2a_ref, wp2_ref, st_ref, b2t_ref,
               out_ref):
    h2 = jnp.maximum(
        jnp.dot(ea_ref[...], w2at_ref[...], preferred_element_type=jnp.float32)
        + b2a_ref[...], 0.0)
    hgt = hgt_ref[...]
    q = lax.dot_general(hgt.astype(jnp.bfloat16), wp2_ref[...],
                        (((0,), (0,)), ((), ())),
                        preferred_element_type=jnp.float32)
    prod = (q * jnp.tile(h2, (1, HID))).astype(jnp.bfloat16)
    mt = lax.dot_general(st_ref[...], prod, (((1,), (1,)), ((), ())),
                         preferred_element_type=jnp.float32)
    c2t = lax.dot_general(b2t_ref[...], hgt, (((1,), (0,)), ((), ())),
                          preferred_element_type=jnp.float32)
    out_ref[...] = mt + c2t


def _node1_body(p_ref, c_ref, xt_ref, r1_ref, b1_ref, out_ref):
    s = jnp.sum(p_ref[...], axis=0)
    cnt = jnp.sum(c_ref[...], axis=0)
    mean = s / jnp.maximum(cnt, 1.0)[None, :]
    xrt = jnp.dot(r1_ref[...], xt_ref[...], preferred_element_type=jnp.float32)
    out_ref[...] = jnp.maximum(mean + xrt + b1_ref[...], 0.0)


def _node2_body(p_ref, c_ref, h1t_ref, r2_ref, b2_ref, wl1_ref, bl1_ref,
                wl2_ref, bl2_ref, out_ref):
    s = jnp.sum(p_ref[...], axis=0)
    cnt = jnp.sum(c_ref[...], axis=0)
    mean = s / jnp.maximum(cnt, 1.0)[None, :]
    h2 = jnp.maximum(
        mean + jnp.dot(r2_ref[...], h1t_ref[...],
                       preferred_element_type=jnp.float32) + b2_ref[...], 0.0)
    h3 = jnp.maximum(
        jnp.dot(wl1_ref[...], h2, preferred_element_type=jnp.float32)
        + bl1_ref[...], 0.0)
    out_ref[...] = jnp.dot(wl2_ref[...], h3,
                           preferred_element_type=jnp.float32) + bl2_ref[...]


def _full(shape):
    nd = len(shape)
    return pl.BlockSpec(shape, lambda i, _n=nd: (0,) * _n)


def kernel(x, edge_index, edge_attr, W1a, b1a, W1b, b1b, root1, bias1,
           W2a, b2a, W2b, b2b, root2, bias2, Wl1, bl1, Wl2, bl2):
    src = edge_index[0]
    dst = edge_index[1]
    zeros8n = jnp.zeros((HID, N), jnp.float32)
    sel = jnp.kron(jnp.eye(HID, dtype=jnp.float32),
                   jnp.ones((1, 64), jnp.float32)).astype(jnp.bfloat16)

    w1at = W1a.T
    wb2 = W1b.reshape(IN_C, HID, 64).reshape(IN_C, HID * 64)
    wb2 = wb2.astype(jnp.bfloat16)
    bb2t = b1b.reshape(IN_C, HID).T
    w2at = W2a.T
    wp2 = W2b.reshape(HID, HID * 64).astype(jnp.bfloat16)
    b2t = b2b.reshape(HID, HID).T
    xt = x.T

    BE = 3200

    xg = _sc_gather_x(x, src)

    m1t = pl.pallas_call(
        _msg1_body,
        grid=(E // BE,),
        in_specs=[
            pl.BlockSpec((BE, 4), lambda i: (i, 0)),
            pl.BlockSpec((BE, IN_C), lambda i: (i, 0)),
            _full((4, 64)), _full((1, 64)),
            _full((IN_C, HID * 64)), _full((HID, HID * 64)),
            _full((HID, IN_C)),
        ],
        out_specs=pl.BlockSpec((HID, BE), lambda i: (0, i)),
        out_shape=jax.ShapeDtypeStruct((HID, E), jnp.float32),
    )(edge_attr, xg, w1at, b1a.reshape(1, 64), wb2, sel, bb2t)

    p1, cnt = _sc_segsum(m1t, dst, zeros8n, True)

    h1t = pl.pallas_call(
        _node1_body,
        out_shape=jax.ShapeDtypeStruct((HID, N), jnp.float32),
    )(p1, cnt, xt, root1, bias1.reshape(HID, 1))

    hgt = _sc_gather_h1(h1t, src)

    m2t = pl.pallas_call(
        _msg2_body,
        grid=(E // BE,),
        in_specs=[
            pl.BlockSpec((BE, 4), lambda i: (i, 0)),
            pl.BlockSpec((HID, BE), lambda i: (0, i)),
            _full((4, 64)), _full((1, 64)),
            _full((HID, HID * 64)), _full((HID, HID * 64)),
            _full((HID, HID)),
        ],
        out_specs=pl.BlockSpec((HID, BE), lambda i: (0, i)),
        out_shape=jax.ShapeDtypeStruct((HID, E), jnp.float32),
    )(edge_attr, hgt, w2at, b2a.reshape(1, 64), wp2, sel, b2t)

    p2 = _sc_segsum(m2t, dst, zeros8n, False)

    out = pl.pallas_call(
        _node2_body,
        out_shape=jax.ShapeDtypeStruct((1, N), jnp.float32),
    )(p2, cnt, h1t, root2, bias2.reshape(HID, 1), Wl1, bl1.reshape(8, 1),
      Wl2, bl2.reshape(1, 1))

    return out[0]
